# Initial kernel scaffold; baseline (speedup 1.0000x reference)
import jax
import jax.numpy as jnp
from jax.experimental import pallas as pl


def kernel(data_NxSxA, data_batched_NxSxA, site_positions_batched_SxSfull, enc_W, site_enc_W, w_rate, W_merge, br_params):
    raise NotImplementedError("write your pallas kernel here")


# speedup vs baseline: 3.3415x; 3.3415x over previous
import numpy as np
import jax
import jax.numpy as jnp
from jax.experimental import pallas as pl
from jax.experimental.pallas import tpu as pltpu

_N = 16
_S = 256
_SFULL = 512
_A = 4
_K = 128
_D = 64
_C = 16
_R = _N - 1
_POOL = _N + _R * _K
_PRIOR_BL = 0.1
_LOG_A = float(np.log(_A))
_CEXP = float(_A) / (_A - 1.0)
_LOG_K = float(np.log(_K))
_LOG_PS = [float(np.log((_N - r) * (_N - r - 1) // 2)) for r in range(_R)]


def _dotx(a, b):
    return jnp.dot(a, b, preferred_element_type=jnp.float32,
                   precision=jax.lax.Precision.HIGHEST)


def _first_argmax_f32(x, iota_row, width):
    mx = jnp.max(x, axis=1, keepdims=True)
    return jnp.min(jnp.where(x == mx, iota_row, float(width)), axis=1,
                   keepdims=True)


def _lse4(blocks):
    m = jnp.maximum(jnp.maximum(blocks[0], blocks[1]),
                    jnp.maximum(blocks[2], blocks[3]))
    s = (jnp.exp(blocks[0] - m) + jnp.exp(blocks[1] - m)
         + jnp.exp(blocks[2] - m) + jnp.exp(blocks[3] - m))
    return m + jnp.log(s)


def _contrib(f_blocks, b, rate_row):
    ex = _CEXP * rate_row * b + 1e-8
    lpd = jnp.log1p(-jnp.exp(-ex)) - _LOG_A
    base = lpd + _lse4(f_blocks)
    return [jnp.logaddexp(base, f_blocks[a] - ex) for a in range(_A)]


def _body(g_ref, i1_ref, i2_ref, leaves_ref, data_ref, encw_ref, sitewt_ref,
          spost_ref, wrate_ref, wm_ref, br_ref,
          logz_ref, ll_ref, m1_ref, m2_ref, b1_ref, b2_ref, embr_ref,
          pf_ref, pe_ref):
    f32 = jnp.float32
    iota_k_col = jax.lax.broadcasted_iota(jnp.int32, (_K, 1), 0).astype(f32)
    iota_k_row = jax.lax.broadcasted_iota(jnp.int32, (1, _K), 1).astype(f32)
    iota_16_row = jax.lax.broadcasted_iota(jnp.int32, (1, 16), 1).astype(f32)
    eye_k = (iota_k_col == iota_k_row).astype(f32)
    ones_row = jnp.ones((1, _K), f32)

    m_cs = jnp.dot(sitewt_ref[...], spost_ref[...],
                   preferred_element_type=f32)
    rate_logit = jnp.dot(wrate_ref[...], m_cs, preferred_element_type=f32)
    rate_row = jax.nn.softplus(rate_logit) + 1e-4
    emb_leaf = jnp.tanh(jnp.dot(data_ref[...], encw_ref[...],
                                preferred_element_type=f32))

    leaves_log = jnp.log(leaves_ref[...])
    lblocks = [leaves_log[:, a * _S:(a + 1) * _S] for a in range(_A)]
    yl = [b - _LOG_A for b in lblocks]
    ml = jnp.maximum(jnp.maximum(yl[0], yl[1]), jnp.maximum(yl[2], yl[3]))
    lse_leaf = ml + jnp.log(jnp.exp(yl[0] - ml) + jnp.exp(yl[1] - ml)
                            + jnp.exp(yl[2] - ml) + jnp.exp(yl[3] - ml))
    ctr_leaf = jnp.sum(lse_leaf, axis=1, keepdims=True)
    s0 = jnp.sum(ctr_leaf)
    pf_ref[0:_N, :] = leaves_log
    pe_ref[0:_N, :] = jnp.concatenate(
        [emb_leaf, ctr_leaf, jnp.zeros((_N, 127 - _D), f32)], axis=1)

    br = br_ref[...]
    br0 = br[0:1, 0:1]
    br1 = br[0:1, 1:2]
    br2 = br[0:1, 2:3]
    br3 = br[0:1, 3:4]
    wm = wm_ref[...]

    mapv = jax.lax.broadcasted_iota(jnp.int32, (_K, 16), 1).astype(f32)
    lp = jnp.zeros((_K, 1), f32)
    lw = jnp.full((_K, 1), -_LOG_K, f32)
    logz = jnp.zeros((1, 1), f32)
    idx_hist = []
    b1_hist = []
    b2_hist = []

    for r in range(_R):
        t = _N - r
        live = _N + r * _K

        lw_row = _dotx(ones_row, eye_k * lw)
        gm = g_ref[r * _K:(r + 1) * _K, :] + lw_row
        idxf = _first_argmax_f32(gm, iota_k_row, _K)
        pm = (iota_k_row == idxf).astype(f32)
        st = _dotx(pm, jnp.concatenate([mapv, lp], axis=1))
        mapv = jnp.round(st[:, 0:16])
        lp = st[:, 16:17]
        idx_hist.append(idxf)

        i1 = i1_ref[:, r:r + 1]
        i2 = i2_ref[:, r:r + 1]
        oh1 = (iota_16_row == i1).astype(f32)
        oh2 = (iota_16_row == i2).astype(f32)
        n1 = jnp.round(jnp.sum(oh1 * mapv, axis=1, keepdims=True))
        n2 = jnp.round(jnp.sum(oh2 * mapv, axis=1, keepdims=True))

        iota_live = jax.lax.broadcasted_iota(jnp.int32, (1, live),
                                             1).astype(f32)
        ohp1 = (iota_live == n1).astype(f32)
        ohp2 = (iota_live == n2).astype(f32)
        ec1 = _dotx(ohp1, pe_ref[0:live, :])
        ec2 = _dotx(ohp2, pe_ref[0:live, :])
        e1 = ec1[:, 0:_D]
        e2 = ec2[:, 0:_D]
        c1v = ec1[:, _D:_D + 1]
        c2v = ec2[:, _D:_D + 1]
        f1 = _dotx(ohp1, pf_ref[0:live, :])
        f2 = _dotx(ohp2, pf_ref[0:live, :])
        f1b = [f1[:, a * _S:(a + 1) * _S] for a in range(_A)]
        f2b = [f2[:, a * _S:(a + 1) * _S] for a in range(_A)]

        diff = e1 - e2
        dist = jnp.sqrt(jnp.sum(diff * diff, axis=1, keepdims=True) + 1e-8)
        b1v = jax.nn.softplus(br0 * dist + br1) * _PRIOR_BL + 1e-4
        b2v = jax.nn.softplus(br2 * dist + br3) * _PRIOR_BL + 1e-4
        e_m = jnp.tanh(jnp.dot(jnp.concatenate([e1, e2], axis=1), wm,
                               preferred_element_type=f32))

        cb1 = _contrib(f1b, b1v, rate_row)
        cb2 = _contrib(f2b, b2v, rate_row)
        mb = [cb1[a] + cb2[a] for a in range(_A)]
        yb = [b - _LOG_A for b in mb]
        mm = jnp.maximum(jnp.maximum(yb[0], yb[1]), jnp.maximum(yb[2], yb[3]))
        lse_m = mm + jnp.log(jnp.exp(yb[0] - mm) + jnp.exp(yb[1] - mm)
                             + jnp.exp(yb[2] - mm) + jnp.exp(yb[3] - mm))
        ctr_m = jnp.sum(lse_m, axis=1, keepdims=True)

        pf_ref[live:live + _K, :] = jnp.concatenate(mb, axis=1)
        pe_ref[live:live + _K, :] = jnp.concatenate(
            [e_m, ctr_m, jnp.zeros((_K, 127 - _D), f32)], axis=1)

        base = s0 if r == 0 else lp
        lp_new = base - c1v - c2v + ctr_m
        log_prior = -2.0 * float(np.log(_PRIOR_BL)) - (b1v + b2v) / _PRIOR_BL
        lw = lp_new - lp + log_prior + _LOG_PS[r]
        lp = lp_new
        mxw = jnp.max(lw)
        logz = logz + (mxw + jnp.log(jnp.sum(jnp.exp(lw - mxw))) - _LOG_K)
        b1_hist.append(b1v)
        b2_hist.append(b2v)

        map_s1 = jnp.concatenate([mapv[:, 1:], mapv[:, :1]], axis=1)
        map_s2 = jnp.concatenate([mapv[:, 2:], mapv[:, :2]], axis=1)
        mapn = jnp.where(iota_16_row < i1, mapv,
                         jnp.where(iota_16_row < i2 - 1.0, map_s1, map_s2))
        newid = float(live) + iota_k_col
        mapv = jnp.where(iota_16_row == float(t - 2), newid, mapn)

    ll_ref[...] = lp
    logz_ref[...] = logz
    ll_row = _dotx(ones_row, eye_k * lp)
    jf = _first_argmax_f32(ll_row, iota_k_row, _K)
    ohj = (iota_k_row == jf).astype(f32)
    for r in range(_R - 1, -1, -1):
        live = _N + r * _K
        m1_ref[0:1, r:r + 1] = _dotx(ohj, i1_ref[:, r:r + 1])
        m2_ref[0:1, r:r + 1] = _dotx(ohj, i2_ref[:, r:r + 1])
        b1_ref[0:1, r:r + 1] = _dotx(ohj, b1_hist[r])
        b2_ref[0:1, r:r + 1] = _dotx(ohj, b2_hist[r])
        embr_ref[r:r + 1, :] = _dotx(ohj, pe_ref[live:live + _K, 0:_D])
        jf = jnp.round(_dotx(ohj, idx_hist[r]))
        ohj = (iota_k_row == jf).astype(f32)


def kernel(data_NxSxA, data_batched_NxSxA, site_positions_batched_SxSfull,
           enc_W, site_enc_W, w_rate, W_merge, br_params):
    f32 = jnp.float32
    kb = jax.random.key(42)
    gs, i1l, i2l = [], [], []
    for r in range(_R):
        t = _N - r
        p = t * (t - 1) // 2
        gs.append(jax.random.gumbel(jax.random.fold_in(kb, 2 * r),
                                    (_K, _K), f32))
        pair = jax.random.randint(jax.random.fold_in(kb, 2 * r + 1),
                                  (_K,), 0, p)
        iu0, iu1 = np.triu_indices(t, 1)
        i1l.append(jnp.asarray(iu0, jnp.int32)[pair])
        i2l.append(jnp.asarray(iu1, jnp.int32)[pair])
    g_flat = jnp.concatenate(gs, axis=0)
    i1 = jnp.pad(jnp.stack(i1l, axis=1).astype(f32), ((0, 0), (0, 16 - _R)))
    i2 = jnp.pad(jnp.stack(i2l, axis=1).astype(f32), ((0, 0), (0, 16 - _R)))
    leaves_t = jnp.transpose(data_batched_NxSxA, (0, 2, 1)).reshape(_N,
                                                                    _A * _S)
    data_flat = data_NxSxA.reshape(_N, _SFULL * _A)
    sitewt = site_enc_W.T
    spost = site_positions_batched_SxSfull.T
    wrow = w_rate.reshape(1, _C)
    brrow = br_params.reshape(1, 4)

    out_shape = [
        jax.ShapeDtypeStruct((1, 1), f32),
        jax.ShapeDtypeStruct((_K, 1), f32),
        jax.ShapeDtypeStruct((1, 16), f32),
        jax.ShapeDtypeStruct((1, 16), f32),
        jax.ShapeDtypeStruct((1, 16), f32),
        jax.ShapeDtypeStruct((1, 16), f32),
        jax.ShapeDtypeStruct((16, _D), f32),
    ]
    scratch = [
        pltpu.VMEM((_POOL, _A * _S), f32),
        pltpu.VMEM((_POOL, 128), f32),
    ]
    logz, ll, m1o, m2o, b1o, b2o, embro = pl.pallas_call(
        _body, out_shape=out_shape, scratch_shapes=scratch,
    )(g_flat, i1, i2, leaves_t, data_flat, enc_W, sitewt, spost, wrow,
      W_merge, brrow)
    return (logz[0, 0], ll[:, 0],
            jnp.round(m1o[0, :_R]).astype(jnp.int32),
            jnp.round(m2o[0, :_R]).astype(jnp.int32),
            b1o[0, :_R], b2o[0, :_R], embro[:_R, :])


# speedup vs baseline: 3.6697x; 1.0982x over previous
import numpy as np
import jax
import jax.numpy as jnp
from jax.experimental import pallas as pl
from jax.experimental.pallas import tpu as pltpu

_N = 16
_S = 256
_SFULL = 512
_A = 4
_K = 128
_D = 64
_C = 16
_R = _N - 1
_POOL = _N + _R * _K
_PRIOR_BL = 0.1
_LOG_A = float(np.log(_A))
_CEXP = float(_A) / (_A - 1.0)
_LOG_K = float(np.log(_K))
_LOG_PS = [float(np.log((_N - r) * (_N - r - 1) // 2)) for r in range(_R)]


def _dotx(a, b):
    return jnp.dot(a, b, preferred_element_type=jnp.float32,
                   precision=jax.lax.Precision.HIGHEST)


def _split3(x):
    hi = x.astype(jnp.bfloat16)
    r1 = x - hi.astype(jnp.float32)
    mid = r1.astype(jnp.bfloat16)
    lo = (r1 - mid.astype(jnp.float32)).astype(jnp.bfloat16)
    return hi, mid, lo


def _gather3(oh_bf, refs, lo_slice, hi_slice):
    acc = jnp.dot(oh_bf, refs[0][lo_slice, hi_slice],
                  preferred_element_type=jnp.float32)
    acc = acc + jnp.dot(oh_bf, refs[1][lo_slice, hi_slice],
                        preferred_element_type=jnp.float32)
    return acc + jnp.dot(oh_bf, refs[2][lo_slice, hi_slice],
                         preferred_element_type=jnp.float32)


def _first_argmax_f32(x, iota_row, width):
    mx = jnp.max(x, axis=1, keepdims=True)
    return jnp.min(jnp.where(x == mx, iota_row, float(width)), axis=1,
                   keepdims=True)


def _lse4(blocks):
    m = jnp.maximum(jnp.maximum(blocks[0], blocks[1]),
                    jnp.maximum(blocks[2], blocks[3]))
    s = (jnp.exp(blocks[0] - m) + jnp.exp(blocks[1] - m)
         + jnp.exp(blocks[2] - m) + jnp.exp(blocks[3] - m))
    return m + jnp.log(s)


def _contrib(f_blocks, b, rate_row):
    ex = _CEXP * rate_row * b + 1e-8
    lpd = jnp.log1p(-jnp.exp(-ex)) - _LOG_A
    base = lpd + _lse4(f_blocks)
    return [jnp.logaddexp(base, f_blocks[a] - ex) for a in range(_A)]


def _body(g_ref, i1_ref, i2_ref, leaves_ref, data_ref, encw_ref, sitewt_ref,
          spost_ref, wrate_ref, wm_ref, br_ref,
          logz_ref, ll_ref, m1_ref, m2_ref, b1_ref, b2_ref, embr_ref,
          pfh_ref, pfm_ref, pfl_ref, peh_ref, pem_ref, pel_ref):
    f32 = jnp.float32
    iota_k_col = jax.lax.broadcasted_iota(jnp.int32, (_K, 1), 0).astype(f32)
    iota_k_row = jax.lax.broadcasted_iota(jnp.int32, (1, _K), 1).astype(f32)
    iota_16_row = jax.lax.broadcasted_iota(jnp.int32, (1, 16), 1).astype(f32)
    eye_k = (iota_k_col == iota_k_row).astype(f32)
    ones_row = jnp.ones((1, _K), f32)

    m_cs = jnp.dot(sitewt_ref[...], spost_ref[...],
                   preferred_element_type=f32)
    rate_logit = jnp.dot(wrate_ref[...], m_cs, preferred_element_type=f32)
    rate_row = jax.nn.softplus(rate_logit) + 1e-4
    emb_leaf = jnp.tanh(jnp.dot(data_ref[...], encw_ref[...],
                                preferred_element_type=f32))

    leaves_log = jnp.log(leaves_ref[...])
    lblocks = [leaves_log[:, a * _S:(a + 1) * _S] for a in range(_A)]
    yl = [b - _LOG_A for b in lblocks]
    ml = jnp.maximum(jnp.maximum(yl[0], yl[1]), jnp.maximum(yl[2], yl[3]))
    lse_leaf = ml + jnp.log(jnp.exp(yl[0] - ml) + jnp.exp(yl[1] - ml)
                            + jnp.exp(yl[2] - ml) + jnp.exp(yl[3] - ml))
    ctr_leaf = jnp.sum(lse_leaf, axis=1, keepdims=True)
    s0 = jnp.sum(ctr_leaf)
    lh, lm, ll3 = _split3(leaves_log)
    pfh_ref[0:_N, :] = lh
    pfm_ref[0:_N, :] = lm
    pfl_ref[0:_N, :] = ll3
    leaf_ec = jnp.concatenate(
        [emb_leaf, ctr_leaf, jnp.zeros((_N, 127 - _D), f32)], axis=1)
    eh, em, el = _split3(leaf_ec)
    peh_ref[0:_N, :] = eh
    pem_ref[0:_N, :] = em
    pel_ref[0:_N, :] = el

    br = br_ref[...]
    br0 = br[0:1, 0:1]
    br1 = br[0:1, 1:2]
    br2 = br[0:1, 2:3]
    br3 = br[0:1, 3:4]
    wm = wm_ref[...]

    mapv = jax.lax.broadcasted_iota(jnp.int32, (_K, 16), 1).astype(f32)
    lp = jnp.zeros((_K, 1), f32)
    lw = jnp.full((_K, 1), -_LOG_K, f32)
    logz = jnp.zeros((1, 1), f32)
    idx_hist = []
    b1_hist = []
    b2_hist = []

    for r in range(_R):
        t = _N - r
        live = _N + r * _K

        lw_row = _dotx(ones_row, eye_k * lw)
        gm = g_ref[r * _K:(r + 1) * _K, :] + lw_row
        idxf = _first_argmax_f32(gm, iota_k_row, _K)
        pm = (iota_k_row == idxf).astype(f32)
        st = _dotx(pm, jnp.concatenate([mapv, lp], axis=1))
        mapv = jnp.round(st[:, 0:16])
        lp = st[:, 16:17]
        idx_hist.append(idxf)

        i1 = i1_ref[:, r:r + 1]
        i2 = i2_ref[:, r:r + 1]
        oh1 = (iota_16_row == i1).astype(f32)
        oh2 = (iota_16_row == i2).astype(f32)
        n1 = jnp.round(jnp.sum(oh1 * mapv, axis=1, keepdims=True))
        n2 = jnp.round(jnp.sum(oh2 * mapv, axis=1, keepdims=True))

        iota_live = jax.lax.broadcasted_iota(jnp.int32, (1, live),
                                             1).astype(f32)
        ohp1 = (iota_live == n1).astype(jnp.bfloat16)
        ohp2 = (iota_live == n2).astype(jnp.bfloat16)
        pe_refs = (peh_ref, pem_ref, pel_ref)
        pf_refs = (pfh_ref, pfm_ref, pfl_ref)
        lsl = slice(0, live)
        asl = slice(None)
        ec1 = _gather3(ohp1, pe_refs, lsl, asl)
        ec2 = _gather3(ohp2, pe_refs, lsl, asl)
        e1 = ec1[:, 0:_D]
        e2 = ec2[:, 0:_D]
        c1v = ec1[:, _D:_D + 1]
        c2v = ec2[:, _D:_D + 1]
        f1 = _gather3(ohp1, pf_refs, lsl, asl)
        f2 = _gather3(ohp2, pf_refs, lsl, asl)
        f1b = [f1[:, a * _S:(a + 1) * _S] for a in range(_A)]
        f2b = [f2[:, a * _S:(a + 1) * _S] for a in range(_A)]

        diff = e1 - e2
        dist = jnp.sqrt(jnp.sum(diff * diff, axis=1, keepdims=True) + 1e-8)
        b1v = jax.nn.softplus(br0 * dist + br1) * _PRIOR_BL + 1e-4
        b2v = jax.nn.softplus(br2 * dist + br3) * _PRIOR_BL + 1e-4
        e_m = jnp.tanh(jnp.dot(jnp.concatenate([e1, e2], axis=1), wm,
                               preferred_element_type=f32))

        cb1 = _contrib(f1b, b1v, rate_row)
        cb2 = _contrib(f2b, b2v, rate_row)
        mb = [cb1[a] + cb2[a] for a in range(_A)]
        yb = [b - _LOG_A for b in mb]
        mm = jnp.maximum(jnp.maximum(yb[0], yb[1]), jnp.maximum(yb[2], yb[3]))
        lse_m = mm + jnp.log(jnp.exp(yb[0] - mm) + jnp.exp(yb[1] - mm)
                             + jnp.exp(yb[2] - mm) + jnp.exp(yb[3] - mm))
        ctr_m = jnp.sum(lse_m, axis=1, keepdims=True)

        mh, mm3, ml3 = _split3(jnp.concatenate(mb, axis=1))
        pfh_ref[live:live + _K, :] = mh
        pfm_ref[live:live + _K, :] = mm3
        pfl_ref[live:live + _K, :] = ml3
        new_ec = jnp.concatenate(
            [e_m, ctr_m, jnp.zeros((_K, 127 - _D), f32)], axis=1)
        nh, nm, nl = _split3(new_ec)
        peh_ref[live:live + _K, :] = nh
        pem_ref[live:live + _K, :] = nm
        pel_ref[live:live + _K, :] = nl

        base = s0 if r == 0 else lp
        lp_new = base - c1v - c2v + ctr_m
        log_prior = -2.0 * float(np.log(_PRIOR_BL)) - (b1v + b2v) / _PRIOR_BL
        lw = lp_new - lp + log_prior + _LOG_PS[r]
        lp = lp_new
        mxw = jnp.max(lw)
        logz = logz + (mxw + jnp.log(jnp.sum(jnp.exp(lw - mxw))) - _LOG_K)
        b1_hist.append(b1v)
        b2_hist.append(b2v)

        map_s1 = jnp.concatenate([mapv[:, 1:], mapv[:, :1]], axis=1)
        map_s2 = jnp.concatenate([mapv[:, 2:], mapv[:, :2]], axis=1)
        mapn = jnp.where(iota_16_row < i1, mapv,
                         jnp.where(iota_16_row < i2 - 1.0, map_s1, map_s2))
        newid = float(live) + iota_k_col
        mapv = jnp.where(iota_16_row == float(t - 2), newid, mapn)

    ll_ref[...] = lp
    logz_ref[...] = logz
    ll_row = _dotx(ones_row, eye_k * lp)
    jf = _first_argmax_f32(ll_row, iota_k_row, _K)
    ohj = (iota_k_row == jf).astype(f32)
    for r in range(_R - 1, -1, -1):
        live = _N + r * _K
        m1_ref[0:1, r:r + 1] = _dotx(ohj, i1_ref[:, r:r + 1])
        m2_ref[0:1, r:r + 1] = _dotx(ohj, i2_ref[:, r:r + 1])
        b1_ref[0:1, r:r + 1] = _dotx(ohj, b1_hist[r])
        b2_ref[0:1, r:r + 1] = _dotx(ohj, b2_hist[r])
        embr_ref[r:r + 1, :] = _gather3(
            ohj.astype(jnp.bfloat16), (peh_ref, pem_ref, pel_ref),
            slice(live, live + _K), slice(0, _D))
        jf = jnp.round(_dotx(ohj, idx_hist[r]))
        ohj = (iota_k_row == jf).astype(f32)


def kernel(data_NxSxA, data_batched_NxSxA, site_positions_batched_SxSfull,
           enc_W, site_enc_W, w_rate, W_merge, br_params):
    f32 = jnp.float32
    kb = jax.random.key(42)
    gs, i1l, i2l = [], [], []
    for r in range(_R):
        t = _N - r
        p = t * (t - 1) // 2
        gs.append(jax.random.gumbel(jax.random.fold_in(kb, 2 * r),
                                    (_K, _K), f32))
        pair = jax.random.randint(jax.random.fold_in(kb, 2 * r + 1),
                                  (_K,), 0, p)
        iu0, iu1 = np.triu_indices(t, 1)
        i1l.append(jnp.asarray(iu0, jnp.int32)[pair])
        i2l.append(jnp.asarray(iu1, jnp.int32)[pair])
    g_flat = jnp.concatenate(gs, axis=0)
    i1 = jnp.pad(jnp.stack(i1l, axis=1).astype(f32), ((0, 0), (0, 16 - _R)))
    i2 = jnp.pad(jnp.stack(i2l, axis=1).astype(f32), ((0, 0), (0, 16 - _R)))
    leaves_t = jnp.transpose(data_batched_NxSxA, (0, 2, 1)).reshape(_N,
                                                                    _A * _S)
    data_flat = data_NxSxA.reshape(_N, _SFULL * _A)
    sitewt = site_enc_W.T
    spost = site_positions_batched_SxSfull.T
    wrow = w_rate.reshape(1, _C)
    brrow = br_params.reshape(1, 4)

    out_shape = [
        jax.ShapeDtypeStruct((1, 1), f32),
        jax.ShapeDtypeStruct((_K, 1), f32),
        jax.ShapeDtypeStruct((1, 16), f32),
        jax.ShapeDtypeStruct((1, 16), f32),
        jax.ShapeDtypeStruct((1, 16), f32),
        jax.ShapeDtypeStruct((1, 16), f32),
        jax.ShapeDtypeStruct((16, _D), f32),
    ]
    bf16 = jnp.bfloat16
    scratch = (
        [pltpu.VMEM((_POOL, _A * _S), bf16) for _ in range(3)]
        + [pltpu.VMEM((_POOL, 128), bf16) for _ in range(3)]
    )
    logz, ll, m1o, m2o, b1o, b2o, embro = pl.pallas_call(
        _body, out_shape=out_shape, scratch_shapes=scratch,
    )(g_flat, i1, i2, leaves_t, data_flat, enc_W, sitewt, spost, wrow,
      W_merge, brrow)
    return (logz[0, 0], ll[:, 0],
            jnp.round(m1o[0, :_R]).astype(jnp.int32),
            jnp.round(m2o[0, :_R]).astype(jnp.int32),
            b1o[0, :_R], b2o[0, :_R], embro[:_R, :])


# speedup vs baseline: 3.7053x; 1.0097x over previous
import numpy as np
import jax
import jax.numpy as jnp
from jax.experimental import pallas as pl
from jax.experimental.pallas import tpu as pltpu

_N = 16
_S = 256
_SFULL = 512
_A = 4
_K = 128
_D = 64
_C = 16
_R = _N - 1
_POOL = _N + _R * _K
_W = _A * _S + _D + 64
_PRIOR_BL = 0.1
_LOG_A = float(np.log(_A))
_CEXP = float(_A) / (_A - 1.0)
_LOG_K = float(np.log(_K))
_LOG_PS = [float(np.log((_N - r) * (_N - r - 1) // 2)) for r in range(_R)]


def _dotx(a, b):
    return jnp.dot(a, b, preferred_element_type=jnp.float32,
                   precision=jax.lax.Precision.HIGHEST)


def _split3(x):
    hi = x.astype(jnp.bfloat16)
    r1 = x - hi.astype(jnp.float32)
    mid = r1.astype(jnp.bfloat16)
    lo = (r1 - mid.astype(jnp.float32)).astype(jnp.bfloat16)
    return hi, mid, lo


def _gather3(oh_bf, refs, rsl, csl):
    acc = jnp.dot(oh_bf, refs[0][rsl, csl],
                  preferred_element_type=jnp.float32)
    acc = acc + jnp.dot(oh_bf, refs[1][rsl, csl],
                        preferred_element_type=jnp.float32)
    return acc + jnp.dot(oh_bf, refs[2][rsl, csl],
                         preferred_element_type=jnp.float32)


def _lse4(blocks):
    m = jnp.maximum(jnp.maximum(blocks[0], blocks[1]),
                    jnp.maximum(blocks[2], blocks[3]))
    s = (jnp.exp(blocks[0] - m) + jnp.exp(blocks[1] - m)
         + jnp.exp(blocks[2] - m) + jnp.exp(blocks[3] - m))
    return m + jnp.log(s)


def _contrib(f_blocks, b, rate_row):
    ex = _CEXP * rate_row * b + 1e-8
    lpd = jnp.log1p(-jnp.exp(-ex)) - _LOG_A
    base = lpd + _lse4(f_blocks)
    return [jnp.logaddexp(base, f_blocks[a] - ex) for a in range(_A)]


def _body(g_ref, i1_ref, i2_ref, leaves_ref, data_ref, encw_ref, sitewt_ref,
          spost_ref, wrate_ref, wm_ref, br_ref,
          logz_ref, ll_ref, m1_ref, m2_ref, b1_ref, b2_ref, embr_ref,
          ph_ref, pm_ref, pl_ref):
    f32 = jnp.float32
    bf16 = jnp.bfloat16
    iota_k_col = jax.lax.broadcasted_iota(jnp.int32, (_K, 1), 0).astype(f32)
    iota_k_row = jax.lax.broadcasted_iota(jnp.int32, (1, _K), 1).astype(f32)
    iota_16_row = jax.lax.broadcasted_iota(jnp.int32, (1, 16), 1).astype(f32)
    iota_pool = jax.lax.broadcasted_iota(jnp.int32, (1, _POOL),
                                         1).astype(f32)
    eye_k = (iota_k_col == iota_k_row).astype(f32)
    ones_row = jnp.ones((1, _K), f32)
    pool_refs = (ph_ref, pm_ref, pl_ref)

    m_cs = jnp.dot(sitewt_ref[...], spost_ref[...],
                   preferred_element_type=f32)
    rate_logit = jnp.dot(wrate_ref[...], m_cs, preferred_element_type=f32)
    rate_row = jax.nn.softplus(rate_logit) + 1e-4
    emb_leaf = jnp.tanh(jnp.dot(data_ref[...], encw_ref[...],
                                preferred_element_type=f32))

    leaves_log = jnp.log(leaves_ref[...])
    yl = [leaves_log[:, a * _S:(a + 1) * _S] - _LOG_A for a in range(_A)]
    ml = jnp.maximum(jnp.maximum(yl[0], yl[1]), jnp.maximum(yl[2], yl[3]))
    lse_leaf = ml + jnp.log(jnp.exp(yl[0] - ml) + jnp.exp(yl[1] - ml)
                            + jnp.exp(yl[2] - ml) + jnp.exp(yl[3] - ml))
    ctr_leaf = jnp.sum(lse_leaf, axis=1, keepdims=True)
    s0 = jnp.sum(ctr_leaf)
    leaf_rows = jnp.concatenate(
        [leaves_log, emb_leaf, ctr_leaf, jnp.zeros((_N, 63), f32)], axis=1)
    lh, lm, ll3 = _split3(leaf_rows)
    ph_ref[0:_N, :] = lh
    pm_ref[0:_N, :] = lm
    pl_ref[0:_N, :] = ll3

    br = br_ref[...]
    br0 = br[0:1, 0:1]
    br1 = br[0:1, 1:2]
    br2 = br[0:1, 2:3]
    br3 = br[0:1, 3:4]
    wm = wm_ref[...]

    mapv = jax.lax.broadcasted_iota(jnp.int32, (_K, 16), 1).astype(f32)
    lp = jnp.zeros((_K, 1), f32)
    lw = jnp.full((_K, 1), -_LOG_K, f32)
    logz = jnp.zeros((1, 1), f32)
    idx_hist = []
    b1_hist = []
    b2_hist = []

    for r in range(_R):
        t = _N - r
        live = _N + r * _K

        gm = g_ref[r * _K:(r + 1) * _K, :] + lw
        mx = jnp.max(gm, axis=0, keepdims=True)
        idx_row = jnp.min(jnp.where(gm == mx, iota_k_col, float(_K)),
                          axis=0, keepdims=True)
        sel = (iota_k_col == idx_row).astype(f32)
        st = jax.lax.dot_general(
            sel, jnp.concatenate([mapv, lp], axis=1),
            dimension_numbers=(((0,), (0,)), ((), ())),
            preferred_element_type=f32,
            precision=jax.lax.Precision.HIGHEST)
        mapv = jnp.round(st[:, 0:16])
        lp = st[:, 16:17]
        idx_hist.append(idx_row)

        i1 = i1_ref[:, r:r + 1]
        i2 = i2_ref[:, r:r + 1]
        oh1 = (iota_16_row == i1).astype(f32)
        oh2 = (iota_16_row == i2).astype(f32)
        n1 = jnp.round(jnp.sum(oh1 * mapv, axis=1, keepdims=True))
        n2 = jnp.round(jnp.sum(oh2 * mapv, axis=1, keepdims=True))

        nstack = jnp.concatenate([n1, n2], axis=0)
        ohs = (iota_pool[:, 0:live] == nstack).astype(bf16)
        gath = _gather3(ohs, pool_refs, slice(0, live), slice(None))
        e1 = gath[0:_K, _A * _S:_A * _S + _D]
        e2 = gath[_K:2 * _K, _A * _S:_A * _S + _D]
        c1v = gath[0:_K, _A * _S + _D:_A * _S + _D + 1]
        c2v = gath[_K:2 * _K, _A * _S + _D:_A * _S + _D + 1]

        diff = e1 - e2
        dist = jnp.sqrt(jnp.sum(diff * diff, axis=1, keepdims=True) + 1e-8)
        b1v = jax.nn.softplus(br0 * dist + br1) * _PRIOR_BL + 1e-4
        b2v = jax.nn.softplus(br2 * dist + br3) * _PRIOR_BL + 1e-4
        e_m = jnp.tanh(jnp.dot(jnp.concatenate([e1, e2], axis=1), wm,
                               preferred_element_type=f32))

        fsb = [gath[:, a * _S:(a + 1) * _S] for a in range(_A)]
        bstack = jnp.concatenate([b1v, b2v], axis=0)
        cb = _contrib(fsb, bstack, rate_row)
        mb = [cb[a][0:_K, :] + cb[a][_K:2 * _K, :] for a in range(_A)]
        yb = [b - _LOG_A for b in mb]
        mm = jnp.maximum(jnp.maximum(yb[0], yb[1]), jnp.maximum(yb[2], yb[3]))
        lse_m = mm + jnp.log(jnp.exp(yb[0] - mm) + jnp.exp(yb[1] - mm)
                             + jnp.exp(yb[2] - mm) + jnp.exp(yb[3] - mm))
        ctr_m = jnp.sum(lse_m, axis=1, keepdims=True)

        new_rows = jnp.concatenate(
            mb + [e_m, ctr_m, jnp.zeros((_K, 63), f32)], axis=1)
        nh, nm, nl = _split3(new_rows)
        ph_ref[live:live + _K, :] = nh
        pm_ref[live:live + _K, :] = nm
        pl_ref[live:live + _K, :] = nl

        base = s0 if r == 0 else lp
        lp_new = base - c1v - c2v + ctr_m
        log_prior = -2.0 * float(np.log(_PRIOR_BL)) - (b1v + b2v) / _PRIOR_BL
        lw = lp_new - lp + log_prior + _LOG_PS[r]
        lp = lp_new
        mxw = jnp.max(lw)
        logz = logz + (mxw + jnp.log(jnp.sum(jnp.exp(lw - mxw))) - _LOG_K)
        b1_hist.append(b1v)
        b2_hist.append(b2v)

        map_s1 = jnp.concatenate([mapv[:, 1:], mapv[:, :1]], axis=1)
        map_s2 = jnp.concatenate([mapv[:, 2:], mapv[:, :2]], axis=1)
        mapn = jnp.where(iota_16_row < i1, mapv,
                         jnp.where(iota_16_row < i2 - 1.0, map_s1, map_s2))
        newid = float(live) + iota_k_col
        mapv = jnp.where(iota_16_row == float(t - 2), newid, mapn)

    ll_ref[...] = lp
    logz_ref[...] = logz
    ll_row = _dotx(ones_row, eye_k * lp)
    mxl = jnp.max(ll_row, axis=1, keepdims=True)
    jf = jnp.min(jnp.where(ll_row == mxl, iota_k_row, float(_K)),
                 axis=1, keepdims=True)
    oh_rows = [None] * _R
    cid_rows = [None] * _R
    for r in range(_R - 1, -1, -1):
        ohj = (iota_k_row == jf).astype(f32)
        oh_rows[r] = ohj
        cid_rows[r] = float(_N + r * _K) + jf
        jf = jnp.round(jnp.sum(ohj * idx_hist[r], axis=1, keepdims=True))
    ohm = jnp.concatenate(oh_rows + [jnp.zeros((1, _K), f32)],
                          axis=0)
    recs = jnp.concatenate(
        [i1_ref[...], i2_ref[...],
         jnp.concatenate(b1_hist, axis=1), jnp.concatenate(b2_hist, axis=1)],
        axis=1)
    y = _dotx(ohm, recs)
    eye16 = (jax.lax.broadcasted_iota(jnp.int32, (16, 16), 0)
             == jax.lax.broadcasted_iota(jnp.int32, (16, 16), 1)).astype(f32)
    d1615 = (jax.lax.broadcasted_iota(jnp.int32, (16, 15), 0)
             == jax.lax.broadcasted_iota(jnp.int32, (16, 15), 1)).astype(f32)
    m1_ref[...] = jnp.sum(y[:, 0:16] * eye16, axis=0, keepdims=True)
    m2_ref[...] = jnp.sum(y[:, 16:32] * eye16, axis=0, keepdims=True)
    b1_ref[0:1, 0:_R] = jnp.sum(y[:, 32:47] * d1615, axis=0, keepdims=True)
    b2_ref[0:1, 0:_R] = jnp.sum(y[:, 47:62] * d1615, axis=0, keepdims=True)
    cids = jnp.concatenate(cid_rows + [jnp.full((1, 1), -1.0, f32)],
                           axis=0)
    ohp = (iota_pool == cids).astype(bf16)
    embr_ref[...] = _gather3(ohp, pool_refs, slice(0, _POOL),
                             slice(_A * _S, _A * _S + _D))


def kernel(data_NxSxA, data_batched_NxSxA, site_positions_batched_SxSfull,
           enc_W, site_enc_W, w_rate, W_merge, br_params):
    f32 = jnp.float32
    kb = jax.random.key(42)
    gs, i1l, i2l = [], [], []
    for r in range(_R):
        t = _N - r
        p = t * (t - 1) // 2
        gs.append(jax.random.gumbel(jax.random.fold_in(kb, 2 * r),
                                    (_K, _K), f32).T)
        pair = jax.random.randint(jax.random.fold_in(kb, 2 * r + 1),
                                  (_K,), 0, p)
        iu0, iu1 = np.triu_indices(t, 1)
        i1l.append(jnp.asarray(iu0, jnp.int32)[pair])
        i2l.append(jnp.asarray(iu1, jnp.int32)[pair])
    g_flat = jnp.concatenate(gs, axis=0)
    i1 = jnp.pad(jnp.stack(i1l, axis=1).astype(f32), ((0, 0), (0, 16 - _R)))
    i2 = jnp.pad(jnp.stack(i2l, axis=1).astype(f32), ((0, 0), (0, 16 - _R)))
    leaves_t = jnp.transpose(data_batched_NxSxA, (0, 2, 1)).reshape(_N,
                                                                    _A * _S)
    data_flat = data_NxSxA.reshape(_N, _SFULL * _A)
    sitewt = site_enc_W.T
    spost = site_positions_batched_SxSfull.T
    wrow = w_rate.reshape(1, _C)
    brrow = br_params.reshape(1, 4)

    out_shape = [
        jax.ShapeDtypeStruct((1, 1), f32),
        jax.ShapeDtypeStruct((_K, 1), f32),
        jax.ShapeDtypeStruct((1, 16), f32),
        jax.ShapeDtypeStruct((1, 16), f32),
        jax.ShapeDtypeStruct((1, 16), f32),
        jax.ShapeDtypeStruct((1, 16), f32),
        jax.ShapeDtypeStruct((16, _D), f32),
    ]
    scratch = [pltpu.VMEM((_POOL, _W), jnp.bfloat16) for _ in range(3)]
    logz, ll, m1o, m2o, b1o, b2o, embro = pl.pallas_call(
        _body, out_shape=out_shape, scratch_shapes=scratch,
    )(g_flat, i1, i2, leaves_t, data_flat, enc_W, sitewt, spost, wrow,
      W_merge, brrow)
    return (logz[0, 0], ll[:, 0],
            jnp.round(m1o[0, :_R]).astype(jnp.int32),
            jnp.round(m2o[0, :_R]).astype(jnp.int32),
            b1o[0, :_R], b2o[0, :_R], embro[:_R, :])


# speedup vs baseline: 3.7545x; 1.0133x over previous
import numpy as np
import jax
import jax.numpy as jnp
from jax.experimental import pallas as pl
from jax.experimental.pallas import tpu as pltpu

_N = 16
_S = 256
_SFULL = 512
_A = 4
_K = 128
_D = 64
_C = 16
_R = _N - 1
_POOL = _N + _R * _K
_W = _A * _S + _S + _D + 64
_MO = _A * _S
_EO = _MO + _S
_CO = _EO + _D
_PRIOR_BL = 0.1
_LOG_A = float(np.log(_A))
_CEXP = float(_A) / (_A - 1.0)
_LOG_K = float(np.log(_K))
_LOG_PS = [float(np.log((_N - r) * (_N - r - 1) // 2)) for r in range(_R)]


def _dotx(a, b):
    return jnp.dot(a, b, preferred_element_type=jnp.float32,
                   precision=jax.lax.Precision.HIGHEST)


def _split3(x):
    hi = x.astype(jnp.bfloat16)
    r1 = x - hi.astype(jnp.float32)
    mid = r1.astype(jnp.bfloat16)
    lo = (r1 - mid.astype(jnp.float32)).astype(jnp.bfloat16)
    return hi, mid, lo


def _gather3(oh_bf, refs, rsl, csl):
    acc = jnp.dot(oh_bf, refs[0][rsl, csl],
                  preferred_element_type=jnp.float32)
    acc = acc + jnp.dot(oh_bf, refs[1][rsl, csl],
                        preferred_element_type=jnp.float32)
    return acc + jnp.dot(oh_bf, refs[2][rsl, csl],
                         preferred_element_type=jnp.float32)


def _max4(blocks):
    return jnp.maximum(jnp.maximum(blocks[0], blocks[1]),
                       jnp.maximum(blocks[2], blocks[3]))


def _body(g_ref, i1_ref, i2_ref, leaves_ref, data_ref, encw_ref, sitewt_ref,
          spost_ref, wrate_ref, wm_ref, br_ref,
          logz_ref, ll_ref, m1_ref, m2_ref, b1_ref, b2_ref, embr_ref,
          ph_ref, pm_ref, pl_ref):
    f32 = jnp.float32
    bf16 = jnp.bfloat16
    iota_k_col = jax.lax.broadcasted_iota(jnp.int32, (_K, 1), 0).astype(f32)
    iota_k_row = jax.lax.broadcasted_iota(jnp.int32, (1, _K), 1).astype(f32)
    iota_16_row = jax.lax.broadcasted_iota(jnp.int32, (1, 16), 1).astype(f32)
    iota_pool = jax.lax.broadcasted_iota(jnp.int32, (1, _POOL),
                                         1).astype(f32)
    eye_k = (iota_k_col == iota_k_row).astype(f32)
    ones_row = jnp.ones((1, _K), f32)
    pool_refs = (ph_ref, pm_ref, pl_ref)

    m_cs = jnp.dot(sitewt_ref[...], spost_ref[...],
                   preferred_element_type=f32)
    rate_logit = jnp.dot(wrate_ref[...], m_cs, preferred_element_type=f32)
    rate_row = jax.nn.softplus(rate_logit) + 1e-4
    emb_leaf = jnp.tanh(jnp.dot(data_ref[...], encw_ref[...],
                                preferred_element_type=f32))

    dleaf = leaves_ref[...]
    dblk = [dleaf[:, a * _S:(a + 1) * _S] for a in range(_A)]
    dmax = _max4(dblk)
    m_leaf = jnp.log(dmax)
    rmax = 1.0 / dmax
    qblk = [d * rmax for d in dblk]
    tq_leaf = qblk[0] + qblk[1] + qblk[2] + qblk[3]
    ctr_leaf = jnp.sum(m_leaf + jnp.log(tq_leaf) - _LOG_A, axis=1,
                       keepdims=True)
    s0 = jnp.sum(ctr_leaf)
    leaf_rows = jnp.concatenate(
        qblk + [m_leaf, emb_leaf, ctr_leaf, jnp.zeros((_N, 63), f32)],
        axis=1)
    lh, lm, ll3 = _split3(leaf_rows)
    ph_ref[0:_N, :] = lh
    pm_ref[0:_N, :] = lm
    pl_ref[0:_N, :] = ll3

    br = br_ref[...]
    br0 = br[0:1, 0:1]
    br1 = br[0:1, 1:2]
    br2 = br[0:1, 2:3]
    br3 = br[0:1, 3:4]
    wm = wm_ref[...]

    mapv = jax.lax.broadcasted_iota(jnp.int32, (_K, 16), 1).astype(f32)
    lp = jnp.zeros((_K, 1), f32)
    lw = jnp.full((_K, 1), -_LOG_K, f32)
    logz = jnp.zeros((1, 1), f32)
    idx_hist = []
    b1_hist = []
    b2_hist = []

    for r in range(_R):
        t = _N - r
        live = _N + r * _K

        gm = g_ref[r * _K:(r + 1) * _K, :] + lw
        mx = jnp.max(gm, axis=0, keepdims=True)
        idx_row = jnp.min(jnp.where(gm == mx, iota_k_col, float(_K)),
                          axis=0, keepdims=True)
        sel = (iota_k_col == idx_row).astype(f32)
        st = jax.lax.dot_general(
            sel, jnp.concatenate([mapv, lp], axis=1),
            dimension_numbers=(((0,), (0,)), ((), ())),
            preferred_element_type=f32,
            precision=jax.lax.Precision.HIGHEST)
        mapv = jnp.round(st[:, 0:16])
        lp = st[:, 16:17]
        idx_hist.append(idx_row)

        i1 = i1_ref[:, r:r + 1]
        i2 = i2_ref[:, r:r + 1]
        oh1 = (iota_16_row == i1).astype(f32)
        oh2 = (iota_16_row == i2).astype(f32)
        n1 = jnp.round(jnp.sum(oh1 * mapv, axis=1, keepdims=True))
        n2 = jnp.round(jnp.sum(oh2 * mapv, axis=1, keepdims=True))

        nstack = jnp.concatenate([n1, n2], axis=0)
        ohs = (iota_pool[:, 0:live] == nstack).astype(bf16)
        gath = _gather3(ohs, pool_refs, slice(0, live), slice(None))
        e1 = gath[0:_K, _EO:_EO + _D]
        e2 = gath[_K:2 * _K, _EO:_EO + _D]
        c1v = gath[0:_K, _CO:_CO + 1]
        c2v = gath[_K:2 * _K, _CO:_CO + 1]

        diff = e1 - e2
        dist = jnp.sqrt(jnp.sum(diff * diff, axis=1, keepdims=True) + 1e-8)
        b1v = jax.nn.softplus(br0 * dist + br1) * _PRIOR_BL + 1e-4
        b2v = jax.nn.softplus(br2 * dist + br3) * _PRIOR_BL + 1e-4
        e_m = jnp.tanh(jnp.dot(jnp.concatenate([e1, e2], axis=1), wm,
                               preferred_element_type=f32))

        qs = [gath[:, a * _S:(a + 1) * _S] for a in range(_A)]
        ms = gath[:, _MO:_MO + _S]
        bstack = jnp.concatenate([b1v, b2v], axis=0)
        exn = jnp.exp(-(_CEXP * rate_row * bstack + 1e-8))
        pd = (1.0 - exn) * (1.0 / _A)
        tq = qs[0] + qs[1] + qs[2] + qs[3]
        pdt = pd * tq
        us = [pdt + exn * q for q in qs]
        prod = [us[a][0:_K, :] * us[a][_K:2 * _K, :] for a in range(_A)]
        m_new0 = ms[0:_K, :] + ms[_K:2 * _K, :]
        nm = _max4(prod)
        rnm = 1.0 / nm
        qn = [p * rnm for p in prod]
        m_new = m_new0 + jnp.log(nm)
        sq = qn[0] + qn[1] + qn[2] + qn[3]
        ctr_m = jnp.sum(m_new + jnp.log(sq) - _LOG_A, axis=1,
                        keepdims=True)

        new_rows = jnp.concatenate(
            qn + [m_new, e_m, ctr_m, jnp.zeros((_K, 63), f32)], axis=1)
        nh, nm, nl = _split3(new_rows)
        ph_ref[live:live + _K, :] = nh
        pm_ref[live:live + _K, :] = nm
        pl_ref[live:live + _K, :] = nl

        base = s0 if r == 0 else lp
        lp_new = base - c1v - c2v + ctr_m
        log_prior = -2.0 * float(np.log(_PRIOR_BL)) - (b1v + b2v) / _PRIOR_BL
        lw = lp_new - lp + log_prior + _LOG_PS[r]
        lp = lp_new
        mxw = jnp.max(lw)
        logz = logz + (mxw + jnp.log(jnp.sum(jnp.exp(lw - mxw))) - _LOG_K)
        b1_hist.append(b1v)
        b2_hist.append(b2v)

        map_s1 = jnp.concatenate([mapv[:, 1:], mapv[:, :1]], axis=1)
        map_s2 = jnp.concatenate([mapv[:, 2:], mapv[:, :2]], axis=1)
        mapn = jnp.where(iota_16_row < i1, mapv,
                         jnp.where(iota_16_row < i2 - 1.0, map_s1, map_s2))
        newid = float(live) + iota_k_col
        mapv = jnp.where(iota_16_row == float(t - 2), newid, mapn)

    ll_ref[...] = lp
    logz_ref[...] = logz
    ll_row = _dotx(ones_row, eye_k * lp)
    mxl = jnp.max(ll_row, axis=1, keepdims=True)
    jf = jnp.min(jnp.where(ll_row == mxl, iota_k_row, float(_K)),
                 axis=1, keepdims=True)
    oh_rows = [None] * _R
    cid_rows = [None] * _R
    for r in range(_R - 1, -1, -1):
        ohj = (iota_k_row == jf).astype(f32)
        oh_rows[r] = ohj
        cid_rows[r] = float(_N + r * _K) + jf
        jf = jnp.round(jnp.sum(ohj * idx_hist[r], axis=1, keepdims=True))
    ohm = jnp.concatenate(oh_rows + [jnp.zeros((1, _K), f32)],
                          axis=0)
    recs = jnp.concatenate(
        [i1_ref[...], i2_ref[...],
         jnp.concatenate(b1_hist, axis=1), jnp.concatenate(b2_hist, axis=1)],
        axis=1)
    y = _dotx(ohm, recs)
    eye16 = (jax.lax.broadcasted_iota(jnp.int32, (16, 16), 0)
             == jax.lax.broadcasted_iota(jnp.int32, (16, 16), 1)).astype(f32)
    d1615 = (jax.lax.broadcasted_iota(jnp.int32, (16, 15), 0)
             == jax.lax.broadcasted_iota(jnp.int32, (16, 15), 1)).astype(f32)
    m1_ref[...] = jnp.sum(y[:, 0:16] * eye16, axis=0, keepdims=True)
    m2_ref[...] = jnp.sum(y[:, 16:32] * eye16, axis=0, keepdims=True)
    b1_ref[0:1, 0:_R] = jnp.sum(y[:, 32:47] * d1615, axis=0, keepdims=True)
    b2_ref[0:1, 0:_R] = jnp.sum(y[:, 47:62] * d1615, axis=0, keepdims=True)
    cids = jnp.concatenate(cid_rows + [jnp.full((1, 1), -1.0, f32)],
                           axis=0)
    ohp = (iota_pool == cids).astype(bf16)
    embr_ref[...] = _gather3(ohp, pool_refs, slice(0, _POOL),
                             slice(_EO, _EO + _D))


def kernel(data_NxSxA, data_batched_NxSxA, site_positions_batched_SxSfull,
           enc_W, site_enc_W, w_rate, W_merge, br_params):
    f32 = jnp.float32
    kb = jax.random.key(42)
    gs, i1l, i2l = [], [], []
    for r in range(_R):
        t = _N - r
        p = t * (t - 1) // 2
        gs.append(jax.random.gumbel(jax.random.fold_in(kb, 2 * r),
                                    (_K, _K), f32).T)
        pair = jax.random.randint(jax.random.fold_in(kb, 2 * r + 1),
                                  (_K,), 0, p)
        iu0, iu1 = np.triu_indices(t, 1)
        i1l.append(jnp.asarray(iu0, jnp.int32)[pair])
        i2l.append(jnp.asarray(iu1, jnp.int32)[pair])
    g_flat = jnp.concatenate(gs, axis=0)
    i1 = jnp.pad(jnp.stack(i1l, axis=1).astype(f32), ((0, 0), (0, 16 - _R)))
    i2 = jnp.pad(jnp.stack(i2l, axis=1).astype(f32), ((0, 0), (0, 16 - _R)))
    leaves_t = jnp.transpose(data_batched_NxSxA, (0, 2, 1)).reshape(_N,
                                                                    _A * _S)
    data_flat = data_NxSxA.reshape(_N, _SFULL * _A)
    sitewt = site_enc_W.T
    spost = site_positions_batched_SxSfull.T
    wrow = w_rate.reshape(1, _C)
    brrow = br_params.reshape(1, 4)

    out_shape = [
        jax.ShapeDtypeStruct((1, 1), f32),
        jax.ShapeDtypeStruct((_K, 1), f32),
        jax.ShapeDtypeStruct((1, 16), f32),
        jax.ShapeDtypeStruct((1, 16), f32),
        jax.ShapeDtypeStruct((1, 16), f32),
        jax.ShapeDtypeStruct((1, 16), f32),
        jax.ShapeDtypeStruct((16, _D), f32),
    ]
    scratch = [pltpu.VMEM((_POOL, _W), jnp.bfloat16) for _ in range(3)]
    logz, ll, m1o, m2o, b1o, b2o, embro = pl.pallas_call(
        _body, out_shape=out_shape, scratch_shapes=scratch,
    )(g_flat, i1, i2, leaves_t, data_flat, enc_W, sitewt, spost, wrow,
      W_merge, brrow)
    return (logz[0, 0], ll[:, 0],
            jnp.round(m1o[0, :_R]).astype(jnp.int32),
            jnp.round(m2o[0, :_R]).astype(jnp.int32),
            b1o[0, :_R], b2o[0, :_R], embro[:_R, :])


# speedup vs baseline: 10.3325x; 2.7520x over previous
import numpy as np
import jax
import jax.numpy as jnp
from jax.experimental import pallas as pl
from jax.experimental.pallas import tpu as pltpu

_N = 16
_S = 256
_SFULL = 512
_A = 4
_K = 128
_D = 64
_C = 16
_R = _N - 1
_POOL = _N + _R * _K
_W = _A * _S + _S + _D + 64
_MO = _A * _S
_EO = _MO + _S
_CO = _EO + _D
_PRIOR_BL = 0.1
_LOG_A = float(np.log(_A))
_CEXP = float(_A) / (_A - 1.0)
_LOG_K = float(np.log(_K))
_LOG_PS = [float(np.log((_N - r) * (_N - r - 1) // 2)) for r in range(_R)]


def _dotx(a, b):
    return jnp.dot(a, b, preferred_element_type=jnp.float32,
                   precision=jax.lax.Precision.HIGHEST)


def _split3(x):
    hi = x.astype(jnp.bfloat16)
    r1 = x - hi.astype(jnp.float32)
    mid = r1.astype(jnp.bfloat16)
    lo = (r1 - mid.astype(jnp.float32)).astype(jnp.bfloat16)
    return hi, mid, lo


def _gather3(oh_bf, refs, rsl, csl):
    acc = jnp.dot(oh_bf, refs[0][rsl, csl],
                  preferred_element_type=jnp.float32)
    acc = acc + jnp.dot(oh_bf, refs[1][rsl, csl],
                        preferred_element_type=jnp.float32)
    return acc + jnp.dot(oh_bf, refs[2][rsl, csl],
                         preferred_element_type=jnp.float32)


def _max4(blocks):
    return jnp.maximum(jnp.maximum(blocks[0], blocks[1]),
                       jnp.maximum(blocks[2], blocks[3]))


def _body(g_ref, i1_ref, i2_ref, leaves_ref, data_ref, encw_ref, sitewt_ref,
          spost_ref, wrate_ref, wm_ref, br_ref,
          logz_ref, ll_ref, m1_ref, m2_ref, b1_ref, b2_ref, embr_ref,
          ph_ref, pm_ref, pl_ref):
    f32 = jnp.float32
    bf16 = jnp.bfloat16
    iota_k_col = jax.lax.broadcasted_iota(jnp.int32, (_K, 1), 0).astype(f32)
    iota_k_row = jax.lax.broadcasted_iota(jnp.int32, (1, _K), 1).astype(f32)
    iota_16_row = jax.lax.broadcasted_iota(jnp.int32, (1, 16), 1).astype(f32)
    iota_pool = jax.lax.broadcasted_iota(jnp.int32, (1, _POOL),
                                         1).astype(f32)
    eye_k = (iota_k_col == iota_k_row).astype(f32)
    ones_row = jnp.ones((1, _K), f32)
    pool_refs = (ph_ref, pm_ref, pl_ref)

    m_cs = jnp.dot(sitewt_ref[...], spost_ref[...],
                   preferred_element_type=f32)
    rate_logit = jnp.dot(wrate_ref[...], m_cs, preferred_element_type=f32)
    rate_row = jax.nn.softplus(rate_logit) + 1e-4
    emb_leaf = jnp.tanh(jnp.dot(data_ref[...], encw_ref[...],
                                preferred_element_type=f32))

    dleaf = leaves_ref[...]
    dblk = [dleaf[:, a * _S:(a + 1) * _S] for a in range(_A)]
    dmax = _max4(dblk)
    m_leaf = jnp.log(dmax)
    rmax = 1.0 / dmax
    qblk = [d * rmax for d in dblk]
    tq_leaf = qblk[0] + qblk[1] + qblk[2] + qblk[3]
    ctr_leaf = jnp.sum(m_leaf + jnp.log(tq_leaf) - _LOG_A, axis=1,
                       keepdims=True)
    s0 = jnp.sum(ctr_leaf)
    leaf_rows = jnp.concatenate(
        qblk + [m_leaf, emb_leaf, ctr_leaf, jnp.zeros((_N, 63), f32)],
        axis=1)
    lh, lm, ll3 = _split3(leaf_rows)
    ph_ref[0:_N, :] = lh
    pm_ref[0:_N, :] = lm
    pl_ref[0:_N, :] = ll3

    br = br_ref[...]
    br0 = br[0:1, 0:1]
    br1 = br[0:1, 1:2]
    br2 = br[0:1, 2:3]
    br3 = br[0:1, 3:4]
    wm = wm_ref[...]

    mapv = jax.lax.broadcasted_iota(jnp.int32, (_K, 16), 1).astype(f32)
    lp = jnp.zeros((_K, 1), f32)
    lw = jnp.full((_K, 1), -_LOG_K, f32)
    logz = jnp.zeros((1, 1), f32)
    idx_hist = []
    b1_hist = []
    b2_hist = []

    for r in range(_R):
        t = _N - r
        live = _N + r * _K

        gm = g_ref[r * _K:(r + 1) * _K, :] + lw
        mx = jnp.max(gm, axis=0, keepdims=True)
        idx_row = jnp.min(jnp.where(gm == mx, iota_k_col, float(_K)),
                          axis=0, keepdims=True)
        sel = (iota_k_col == idx_row).astype(f32)
        st = jax.lax.dot_general(
            sel, jnp.concatenate([mapv, lp], axis=1),
            dimension_numbers=(((0,), (0,)), ((), ())),
            preferred_element_type=f32,
            precision=jax.lax.Precision.HIGHEST)
        mapv = jnp.round(st[:, 0:16])
        lp = st[:, 16:17]
        idx_hist.append(idx_row)

        i1 = i1_ref[:, r:r + 1]
        i2 = i2_ref[:, r:r + 1]
        oh1 = (iota_16_row == i1).astype(f32)
        oh2 = (iota_16_row == i2).astype(f32)
        n1 = jnp.round(jnp.sum(oh1 * mapv, axis=1, keepdims=True))
        n2 = jnp.round(jnp.sum(oh2 * mapv, axis=1, keepdims=True))

        nstack = jnp.concatenate([n1, n2], axis=0)
        ohs = (iota_pool[:, 0:live] == nstack).astype(bf16)
        gath = _gather3(ohs, pool_refs, slice(0, live), slice(None))
        e1 = gath[0:_K, _EO:_EO + _D]
        e2 = gath[_K:2 * _K, _EO:_EO + _D]
        c1v = gath[0:_K, _CO:_CO + 1]
        c2v = gath[_K:2 * _K, _CO:_CO + 1]

        diff = e1 - e2
        dist = jnp.sqrt(jnp.sum(diff * diff, axis=1, keepdims=True) + 1e-8)
        b1v = jax.nn.softplus(br0 * dist + br1) * _PRIOR_BL + 1e-4
        b2v = jax.nn.softplus(br2 * dist + br3) * _PRIOR_BL + 1e-4
        e_m = jnp.tanh(jnp.dot(jnp.concatenate([e1, e2], axis=1), wm,
                               preferred_element_type=f32))

        qs = [gath[:, a * _S:(a + 1) * _S] for a in range(_A)]
        ms = gath[:, _MO:_MO + _S]
        bstack = jnp.concatenate([b1v, b2v], axis=0)
        exn = jnp.exp(-(_CEXP * rate_row * bstack + 1e-8))
        pd = (1.0 - exn) * (1.0 / _A)
        tq = qs[0] + qs[1] + qs[2] + qs[3]
        pdt = pd * tq
        us = [pdt + exn * q for q in qs]
        prod = [us[a][0:_K, :] * us[a][_K:2 * _K, :] for a in range(_A)]
        m_new0 = ms[0:_K, :] + ms[_K:2 * _K, :]
        nm = _max4(prod)
        rnm = 1.0 / nm
        qn = [p * rnm for p in prod]
        m_new = m_new0 + jnp.log(nm)
        sq = qn[0] + qn[1] + qn[2] + qn[3]
        ctr_m = jnp.sum(m_new + jnp.log(sq) - _LOG_A, axis=1,
                        keepdims=True)

        new_rows = jnp.concatenate(
            qn + [m_new, e_m, ctr_m, jnp.zeros((_K, 63), f32)], axis=1)
        nh, nm, nl = _split3(new_rows)
        ph_ref[live:live + _K, :] = nh
        pm_ref[live:live + _K, :] = nm
        pl_ref[live:live + _K, :] = nl

        base = s0 if r == 0 else lp
        lp_new = base - c1v - c2v + ctr_m
        log_prior = -2.0 * float(np.log(_PRIOR_BL)) - (b1v + b2v) / _PRIOR_BL
        lw = lp_new - lp + log_prior + _LOG_PS[r]
        lp = lp_new
        mxw = jnp.max(lw)
        logz = logz + (mxw + jnp.log(jnp.sum(jnp.exp(lw - mxw))) - _LOG_K)
        b1_hist.append(b1v)
        b2_hist.append(b2v)

        map_s1 = jnp.concatenate([mapv[:, 1:], mapv[:, :1]], axis=1)
        map_s2 = jnp.concatenate([mapv[:, 2:], mapv[:, :2]], axis=1)
        mapn = jnp.where(iota_16_row < i1, mapv,
                         jnp.where(iota_16_row < i2 - 1.0, map_s1, map_s2))
        newid = float(live) + iota_k_col
        mapv = jnp.where(iota_16_row == float(t - 2), newid, mapn)

    ll_ref[...] = lp
    logz_ref[...] = logz
    ll_row = _dotx(ones_row, eye_k * lp)
    mxl = jnp.max(ll_row, axis=1, keepdims=True)
    jf = jnp.min(jnp.where(ll_row == mxl, iota_k_row, float(_K)),
                 axis=1, keepdims=True)
    oh_rows = [None] * _R
    cid_rows = [None] * _R
    for r in range(_R - 1, -1, -1):
        ohj = (iota_k_row == jf).astype(f32)
        oh_rows[r] = ohj
        cid_rows[r] = float(_N + r * _K) + jf
        jf = jnp.round(jnp.sum(ohj * idx_hist[r], axis=1, keepdims=True))
    ohm = jnp.concatenate(oh_rows + [jnp.zeros((1, _K), f32)],
                          axis=0)
    recs = jnp.concatenate(
        [i1_ref[...], i2_ref[...],
         jnp.concatenate(b1_hist, axis=1), jnp.concatenate(b2_hist, axis=1)],
        axis=1)
    y = _dotx(ohm, recs)
    eye16 = (jax.lax.broadcasted_iota(jnp.int32, (16, 16), 0)
             == jax.lax.broadcasted_iota(jnp.int32, (16, 16), 1)).astype(f32)
    d1615 = (jax.lax.broadcasted_iota(jnp.int32, (16, 15), 0)
             == jax.lax.broadcasted_iota(jnp.int32, (16, 15), 1)).astype(f32)
    m1_ref[...] = jnp.sum(y[:, 0:16] * eye16, axis=0, keepdims=True)
    m2_ref[...] = jnp.sum(y[:, 16:32] * eye16, axis=0, keepdims=True)
    b1_ref[0:1, 0:_R] = jnp.sum(y[:, 32:47] * d1615, axis=0, keepdims=True)
    b2_ref[0:1, 0:_R] = jnp.sum(y[:, 47:62] * d1615, axis=0, keepdims=True)
    cids = jnp.concatenate(cid_rows + [jnp.full((1, 1), -1.0, f32)],
                           axis=0)
    ohp = (iota_pool == cids).astype(bf16)
    embr_ref[...] = _gather3(ohp, pool_refs, slice(0, _POOL),
                             slice(_EO, _EO + _D))


def kernel(data_NxSxA, data_batched_NxSxA, site_positions_batched_SxSfull,
           enc_W, site_enc_W, w_rate, W_merge, br_params):
    f32 = jnp.float32
    kb = jax.random.key(42)
    rr = jnp.arange(_R)
    gkeys = jax.vmap(lambda i: jax.random.fold_in(kb, i))(2 * rr)
    pkeys = jax.vmap(lambda i: jax.random.fold_in(kb, i))(2 * rr + 1)
    gall = jax.vmap(lambda k: jax.random.gumbel(k, (_K, _K), f32))(gkeys)
    g_flat = jnp.transpose(gall, (0, 2, 1)).reshape(_R * _K, _K)
    ps = jnp.asarray([(_N - r) * (_N - r - 1) // 2 for r in range(_R)],
                     jnp.int32)
    pairs = jax.vmap(lambda k, p: jax.random.randint(k, (_K,), 0, p))(
        pkeys, ps)
    pmax = _N * (_N - 1) // 2
    tu0 = np.zeros((_R, pmax), np.int32)
    tu1 = np.zeros((_R, pmax), np.int32)
    for r in range(_R):
        a, b = np.triu_indices(_N - r, 1)
        tu0[r, :len(a)] = a
        tu1[r, :len(b)] = b
    i1m = jnp.take_along_axis(jnp.asarray(tu0), pairs, axis=1).T
    i2m = jnp.take_along_axis(jnp.asarray(tu1), pairs, axis=1).T
    i1 = jnp.pad(i1m.astype(f32), ((0, 0), (0, 16 - _R)))
    i2 = jnp.pad(i2m.astype(f32), ((0, 0), (0, 16 - _R)))
    leaves_t = jnp.transpose(data_batched_NxSxA, (0, 2, 1)).reshape(_N,
                                                                    _A * _S)
    data_flat = data_NxSxA.reshape(_N, _SFULL * _A)
    sitewt = site_enc_W.T
    spost = site_positions_batched_SxSfull.T
    wrow = w_rate.reshape(1, _C)
    brrow = br_params.reshape(1, 4)

    out_shape = [
        jax.ShapeDtypeStruct((1, 1), f32),
        jax.ShapeDtypeStruct((_K, 1), f32),
        jax.ShapeDtypeStruct((1, 16), f32),
        jax.ShapeDtypeStruct((1, 16), f32),
        jax.ShapeDtypeStruct((1, 16), f32),
        jax.ShapeDtypeStruct((1, 16), f32),
        jax.ShapeDtypeStruct((16, _D), f32),
    ]
    scratch = [pltpu.VMEM((_POOL, _W), jnp.bfloat16) for _ in range(3)]
    logz, ll, m1o, m2o, b1o, b2o, embro = pl.pallas_call(
        _body, out_shape=out_shape, scratch_shapes=scratch,
    )(g_flat, i1, i2, leaves_t, data_flat, enc_W, sitewt, spost, wrow,
      W_merge, brrow)
    return (logz[0, 0], ll[:, 0],
            jnp.round(m1o[0, :_R]).astype(jnp.int32),
            jnp.round(m2o[0, :_R]).astype(jnp.int32),
            b1o[0, :_R], b2o[0, :_R], embro[:_R, :])
